# baseline (speedup 1.0000x reference)
import functools

import jax
import jax.numpy as jnp
from jax import lax
from jax.experimental import pallas as pl
from jax.experimental.pallas import tpu as pltpu
from jax.experimental.pallas import tpu_sc as plsc

DECAY = 0.3
THRESHOLD = 0.9
CH = 40
NW = 32


def _make_stream_s(n, d, n_class, ns):
    rows_w = ns // NW
    n_chunks = rows_w // CH
    mesh = plsc.VectorSubcoreMesh(core_axis_name="c", subcore_axis_name="s")

    @functools.partial(
        pl.kernel, mesh=mesh,
        out_type=jax.ShapeDtypeStruct((NW, n_class + 1, d), jnp.float32),
        scratch_types=[
            pltpu.VMEM((2, CH, n_class), jnp.float32),
            pltpu.VMEM((2, CH, d), jnp.float32),
            pltpu.VMEM((n_class + 1, d), jnp.float32),
            pltpu.SemaphoreType.DMA,
            pltpu.SemaphoreType.DMA,
        ],
    )
    def stream_s(y_hbm, feat_hbm, part_hbm, ybuf, fbuf, accum, sem0, sem1):
        cid = lax.axis_index("c")
        sid = lax.axis_index("s")
        wid = sid * 2 + cid
        base = wid * rows_w
        sems = (sem0, sem1)

        def zrow(i, _):
            for k in range(d // 16):
                accum[i, pl.ds(k * 16, 16)] = jnp.zeros((16,), jnp.float32)
            return 0
        lax.fori_loop(0, n_class + 1, zrow, 0)

        for b in range(2):
            pltpu.async_copy(y_hbm.at[pl.ds(base + b * CH, CH)],
                             ybuf.at[b], sems[b])
            pltpu.async_copy(feat_hbm.at[pl.ds(base + b * CH, CH)],
                             fbuf.at[b], sems[b])

        iota16 = lax.iota(jnp.int32, 16)
        dnums = lax.GatherDimensionNumbers(
            offset_dims=(), collapsed_slice_dims=(0,), start_index_map=(0,))

        def shuf(v, idx):
            return lax.gather(v, idx.reshape(16, 1), dnums, (1,),
                              mode=lax.GatherScatterMode.PROMISE_IN_BOUNDS)

        perms = [jnp.bitwise_xor(iota16, s) for s in (8, 4, 2, 1)]

        def tree(op, xs):
            while len(xs) > 1:
                nxt = [op(xs[i], xs[i + 1]) for i in range(0, len(xs) - 1, 2)]
                if len(xs) % 2:
                    nxt.append(xs[-1])
                xs = nxt
            return xs[0]

        def one_row(buf, r):
            yv = [ybuf[buf, r, pl.ds(k * 16, 16)]
                  for k in range(n_class // 16)]
            m = tree(jnp.maximum, yv)
            for p in perms:
                m = jnp.maximum(m, shuf(m, p))
            cands = [jnp.where(yv[k] == m, iota16 + (16 * k), n_class)
                     for k in range(len(yv))]
            c = tree(jnp.minimum, cands)
            for p in perms:
                c = jnp.minimum(c, shuf(c, p))
            w = jnp.where(m > THRESHOLD, m, 0.0)
            return c[0], w

        def compute_chunk(buf):
            def pair_body(r2, _):
                rows = [r2 * 2, r2 * 2 + 1]
                labs_ws = [one_row(buf, r) for r in rows]
                for r, (lab, w) in zip(rows, labs_ws):
                    fv = [fbuf[buf, r, pl.ds(k * 16, 16)]
                          for k in range(d // 16)]
                    prods = [v * w for v in fv]
                    for k in range(d // 16):
                        plsc.addupdate(accum.at[lab, pl.ds(k * 16, 16)],
                                       prods[k])
                    e_lane = jnp.where(iota16 == (lab & 15), 1.0, 0.0)
                    plsc.addupdate(
                        accum.at[n_class, pl.ds((lab >> 4) * 16, 16)],
                        e_lane)
                return 0
            lax.fori_loop(0, CH // 2, pair_body, 0)

        def outer(jp, _):
            for b in range(2):
                j = jp * 2 + b

                @pl.when(j < n_chunks)
                def _do():
                    pltpu.make_async_copy(
                        y_hbm.at[pl.ds(base + j * CH, CH)],
                        ybuf.at[b], sems[b]).wait()
                    pltpu.make_async_copy(
                        feat_hbm.at[pl.ds(base + j * CH, CH)],
                        fbuf.at[b], sems[b]).wait()
                    compute_chunk(b)

                    @pl.when(j + 2 < n_chunks)
                    def _next():
                        pltpu.async_copy(
                            y_hbm.at[pl.ds(base + (j + 2) * CH, CH)],
                            ybuf.at[b], sems[b])
                        pltpu.async_copy(
                            feat_hbm.at[pl.ds(base + (j + 2) * CH, CH)],
                            fbuf.at[b], sems[b])
            return 0

        lax.fori_loop(0, (n_chunks + 1) // 2, outer, 0)
        pltpu.sync_copy(accum, part_hbm.at[wid])

    return stream_s


def _tc_reduce_body(yt_ref, f_ref, sum_ref, tcnt_ref, *, n_steps, n_class):
    i = pl.program_id(0)

    @pl.when(i == 0)
    def _init():
        sum_ref[...] = jnp.zeros_like(sum_ref)
        tcnt_ref[...] = jnp.zeros_like(tcnt_ref)

    y = yt_ref[...]
    bn = y.shape[0]
    m = jnp.max(y, axis=1, keepdims=True)
    iota = lax.broadcasted_iota(jnp.int32, (bn, n_class), 1)
    eq = y >= m
    first = jnp.min(jnp.where(eq, iota, n_class), axis=1, keepdims=True)
    onehot = iota == first
    sel = jnp.where(m > THRESHOLD, m, 0.0)
    w = jnp.where(onehot, sel, 0.0)
    tcnt_ref[...] += jnp.sum(onehot.astype(jnp.float32), axis=0,
                             keepdims=True)
    sum_ref[...] += jax.lax.dot_general(
        w.astype(jnp.bfloat16), f_ref[...].astype(jnp.bfloat16),
        (((0,), (0,)), ((), ())),
        preferred_element_type=jnp.float32)


def _tc_reduce(yt, f, bn=3200, blk0=0, n_steps=None):
    n, n_class = yt.shape
    d = f.shape[1]
    if n_steps is None:
        n_steps = n // bn
    row_spec = pl.BlockSpec((bn, n_class), lambda i: (i + blk0, 0))
    return pl.pallas_call(
        functools.partial(_tc_reduce_body, n_steps=n_steps, n_class=n_class),
        grid=(n_steps,),
        in_specs=[row_spec, row_spec],
        out_specs=[
            pl.BlockSpec((n_class, d), lambda i: (0, 0)),
            pl.BlockSpec((1, n_class), lambda i: (0, 0)),
        ],
        out_shape=[
            jax.ShapeDtypeStruct((n_class, d), jnp.float32),
            jax.ShapeDtypeStruct((1, n_class), jnp.float32),
        ],
        compiler_params=pltpu.CompilerParams(
            dimension_semantics=("arbitrary",)),
    )(yt, f)


def _combine_body(sp_ref, stsum_ref, stcnt_ref, tsum_ref, tcnt_ref, sc_ref,
                  tc_ref, loss_ref, *, d):
    nw, rows, _ = sp_ref.shape
    n_class = rows - 1
    s_acc = sp_ref[0]
    for w in range(1, nw):
        s_acc = s_acc + sp_ref[w]
    s_sum = s_acc[:n_class, :] + stsum_ref[...]
    cnt_row = s_acc[n_class:, :] + stcnt_ref[...]
    iota_r = lax.broadcasted_iota(jnp.int32, (n_class, n_class), 0)
    iota_c = lax.broadcasted_iota(jnp.int32, (n_class, n_class), 1)
    eye = (iota_r == iota_c).astype(jnp.float32)
    cnt_col = jax.lax.dot_general(eye, cnt_row, (((1,), (1,)), ((), ())),
                                  preferred_element_type=jnp.float32)
    s_n = jnp.maximum(cnt_col, 1.0)
    t_n = jnp.maximum(tcnt_ref[...], 1.0)
    cur_s = s_sum / s_n
    cur_t = tsum_ref[...] / t_n
    s_c = (1.0 - DECAY) * sc_ref[...] + DECAY * cur_s
    t_c = (1.0 - DECAY) * tc_ref[...] + DECAY * cur_t
    sq = (s_c - t_c) ** 2
    total = jnp.sum(jnp.sum(sq, axis=1, keepdims=True), axis=0, keepdims=True)
    loss_ref[...] = total / float(sq.shape[0] * sq.shape[1])


def kernel(s_feature, t_feature, y_s, y_t, s_centroid, t_centroid):
    n, d = s_feature.shape
    n_class = y_s.shape[1]

    if n == 160000:
        ns, bn_tail, blk0, tail_steps = 140800, 1280, 110, 15
    else:
        ns, bn_tail = n, None

    s_part = _make_stream_s(n, d, n_class, ns)(y_s, s_feature)
    t_sum, t_cnt = _tc_reduce(y_t, t_feature)
    if bn_tail is not None:
        st_sum, st_cnt = _tc_reduce(y_s, s_feature, bn=bn_tail, blk0=blk0,
                                    n_steps=tail_steps)
    else:
        st_sum = jnp.zeros((n_class, d), jnp.float32)
        st_cnt = jnp.zeros((1, n_class), jnp.float32)

    loss = pl.pallas_call(
        functools.partial(_combine_body, d=d),
        out_shape=jax.ShapeDtypeStruct((1, 1), jnp.float32),
    )(s_part, st_sum, st_cnt, t_sum,
      t_cnt.reshape(n_class, 1), s_centroid, t_centroid)
    return loss[0, 0]


# speedup vs baseline: 1.0106x; 1.0106x over previous
import functools

import jax
import jax.numpy as jnp
from jax import lax
from jax.experimental import pallas as pl
from jax.experimental.pallas import tpu as pltpu
from jax.experimental.pallas import tpu_sc as plsc

DECAY = 0.3
THRESHOLD = 0.9
CH = 40
NW = 32


def _make_stream_s(n, d, n_class, ns):
    rows_w = ns // NW
    n_chunks = rows_w // CH
    mesh = plsc.VectorSubcoreMesh(core_axis_name="c", subcore_axis_name="s")

    @functools.partial(
        pl.kernel, mesh=mesh,
        out_type=jax.ShapeDtypeStruct((NW, n_class + 1, d), jnp.float32),
        scratch_types=[
            pltpu.VMEM((2, CH, n_class), jnp.float32),
            pltpu.VMEM((2, CH, d), jnp.float32),
            pltpu.VMEM((n_class + 1, d), jnp.float32),
            pltpu.SemaphoreType.DMA,
            pltpu.SemaphoreType.DMA,
        ],
    )
    def stream_s(y_hbm, feat_hbm, part_hbm, ybuf, fbuf, accum, sem0, sem1):
        cid = lax.axis_index("c")
        sid = lax.axis_index("s")
        wid = sid * 2 + cid
        base = wid * rows_w
        sems = (sem0, sem1)

        def zrow(i, _):
            for k in range(d // 16):
                accum[i, pl.ds(k * 16, 16)] = jnp.zeros((16,), jnp.float32)
            return 0
        lax.fori_loop(0, n_class + 1, zrow, 0)

        for b in range(2):
            pltpu.async_copy(y_hbm.at[pl.ds(base + b * CH, CH)],
                             ybuf.at[b], sems[b])
            pltpu.async_copy(feat_hbm.at[pl.ds(base + b * CH, CH)],
                             fbuf.at[b], sems[b])

        iota16 = lax.iota(jnp.int32, 16)
        dnums = lax.GatherDimensionNumbers(
            offset_dims=(), collapsed_slice_dims=(0,), start_index_map=(0,))

        def shuf(v, idx):
            return lax.gather(v, idx.reshape(16, 1), dnums, (1,),
                              mode=lax.GatherScatterMode.PROMISE_IN_BOUNDS)

        perms = [jnp.bitwise_xor(iota16, s) for s in (8, 4, 2, 1)]

        def tree(op, xs):
            while len(xs) > 1:
                nxt = [op(xs[i], xs[i + 1]) for i in range(0, len(xs) - 1, 2)]
                if len(xs) % 2:
                    nxt.append(xs[-1])
                xs = nxt
            return xs[0]

        def one_row(buf, r):
            yv = [ybuf[buf, r, pl.ds(k * 16, 16)]
                  for k in range(n_class // 16)]
            m = tree(jnp.maximum, yv)
            for p in perms:
                m = jnp.maximum(m, shuf(m, p))
            cands = [jnp.where(yv[k] == m, iota16 + (16 * k), n_class)
                     for k in range(len(yv))]
            c = tree(jnp.minimum, cands)
            for p in perms:
                c = jnp.minimum(c, shuf(c, p))
            w = jnp.where(m > THRESHOLD, m, 0.0)
            return c[0], w

        def compute_chunk(buf):
            def pair_body(r2, _):
                rows = [r2 * 2, r2 * 2 + 1]
                labs_ws = [one_row(buf, r) for r in rows]
                for r, (lab, w) in zip(rows, labs_ws):
                    fv = [fbuf[buf, r, pl.ds(k * 16, 16)]
                          for k in range(d // 16)]
                    prods = [v * w for v in fv]
                    for k in range(d // 16):
                        plsc.addupdate(accum.at[lab, pl.ds(k * 16, 16)],
                                       prods[k])
                    e_lane = jnp.where(iota16 == (lab & 15), 1.0, 0.0)
                    plsc.addupdate(
                        accum.at[n_class, pl.ds((lab >> 4) * 16, 16)],
                        e_lane)
                return 0
            lax.fori_loop(0, CH // 2, pair_body, 0)

        def outer(jp, _):
            for b in range(2):
                j = jp * 2 + b

                @pl.when(j < n_chunks)
                def _do():
                    pltpu.make_async_copy(
                        y_hbm.at[pl.ds(base + j * CH, CH)],
                        ybuf.at[b], sems[b]).wait()
                    pltpu.make_async_copy(
                        feat_hbm.at[pl.ds(base + j * CH, CH)],
                        fbuf.at[b], sems[b]).wait()
                    compute_chunk(b)

                    @pl.when(j + 2 < n_chunks)
                    def _next():
                        pltpu.async_copy(
                            y_hbm.at[pl.ds(base + (j + 2) * CH, CH)],
                            ybuf.at[b], sems[b])
                        pltpu.async_copy(
                            feat_hbm.at[pl.ds(base + (j + 2) * CH, CH)],
                            fbuf.at[b], sems[b])
            return 0

        lax.fori_loop(0, (n_chunks + 1) // 2, outer, 0)
        pltpu.sync_copy(accum, part_hbm.at[wid])

    return stream_s


def _tc_reduce_body(yt_ref, f_ref, sum_ref, tcnt_ref, *, n_steps, n_class):
    i = pl.program_id(0)

    @pl.when(i == 0)
    def _init():
        sum_ref[...] = jnp.zeros_like(sum_ref)
        tcnt_ref[...] = jnp.zeros_like(tcnt_ref)

    y = yt_ref[...]
    bn = y.shape[0]
    m = jnp.max(y, axis=1, keepdims=True)
    iota = lax.broadcasted_iota(jnp.int32, (bn, n_class), 1)
    eq = y >= m
    first = jnp.min(jnp.where(eq, iota, n_class), axis=1, keepdims=True)
    onehot = iota == first
    sel = jnp.where(m > THRESHOLD, m, 0.0)
    w = jnp.where(onehot, sel, 0.0)
    tcnt_ref[...] += jnp.sum(onehot.astype(jnp.float32), axis=0,
                             keepdims=True)
    sum_ref[...] += jax.lax.dot_general(
        w.astype(jnp.bfloat16), f_ref[...].astype(jnp.bfloat16),
        (((0,), (0,)), ((), ())),
        preferred_element_type=jnp.float32)


def _tc_reduce(yt, f, bn=3200, blk0=0, n_steps=None):
    n, n_class = yt.shape
    d = f.shape[1]
    if n_steps is None:
        n_steps = n // bn
    row_spec = pl.BlockSpec((bn, n_class), lambda i: (i + blk0, 0))
    return pl.pallas_call(
        functools.partial(_tc_reduce_body, n_steps=n_steps, n_class=n_class),
        grid=(n_steps,),
        in_specs=[row_spec, row_spec],
        out_specs=[
            pl.BlockSpec((n_class, d), lambda i: (0, 0)),
            pl.BlockSpec((1, n_class), lambda i: (0, 0)),
        ],
        out_shape=[
            jax.ShapeDtypeStruct((n_class, d), jnp.float32),
            jax.ShapeDtypeStruct((1, n_class), jnp.float32),
        ],
        compiler_params=pltpu.CompilerParams(
            dimension_semantics=("arbitrary",)),
    )(yt, f)


def _combine_body(sp_ref, stsum_ref, stcnt_ref, tsum_ref, tcnt_ref, sc_ref,
                  tc_ref, loss_ref, *, d):
    nw, rows, _ = sp_ref.shape
    n_class = rows - 1
    s_acc = sp_ref[0]
    for w in range(1, nw):
        s_acc = s_acc + sp_ref[w]
    s_sum = s_acc[:n_class, :] + stsum_ref[...]
    cnt_row = s_acc[n_class:, :] + stcnt_ref[...]
    iota_r = lax.broadcasted_iota(jnp.int32, (n_class, n_class), 0)
    iota_c = lax.broadcasted_iota(jnp.int32, (n_class, n_class), 1)
    eye = (iota_r == iota_c).astype(jnp.float32)
    cnt_col = jax.lax.dot_general(eye, cnt_row, (((1,), (1,)), ((), ())),
                                  preferred_element_type=jnp.float32)
    s_n = jnp.maximum(cnt_col, 1.0)
    t_n = jnp.maximum(tcnt_ref[...], 1.0)
    cur_s = s_sum / s_n
    cur_t = tsum_ref[...] / t_n
    s_c = (1.0 - DECAY) * sc_ref[...] + DECAY * cur_s
    t_c = (1.0 - DECAY) * tc_ref[...] + DECAY * cur_t
    sq = (s_c - t_c) ** 2
    total = jnp.sum(jnp.sum(sq, axis=1, keepdims=True), axis=0, keepdims=True)
    loss_ref[...] = total / float(sq.shape[0] * sq.shape[1])


def kernel(s_feature, t_feature, y_s, y_t, s_centroid, t_centroid):
    n, d = s_feature.shape
    n_class = y_s.shape[1]

    if n == 160000:
        ns, bn_tail, blk0, tail_steps = 139520, 1280, 109, 16
    else:
        ns, bn_tail = n, None

    s_part = _make_stream_s(n, d, n_class, ns)(y_s, s_feature)
    t_sum, t_cnt = _tc_reduce(y_t, t_feature)
    if bn_tail is not None:
        st_sum, st_cnt = _tc_reduce(y_s, s_feature, bn=bn_tail, blk0=blk0,
                                    n_steps=tail_steps)
    else:
        st_sum = jnp.zeros((n_class, d), jnp.float32)
        st_cnt = jnp.zeros((1, n_class), jnp.float32)

    loss = pl.pallas_call(
        functools.partial(_combine_body, d=d),
        out_shape=jax.ShapeDtypeStruct((1, 1), jnp.float32),
    )(s_part, st_sum, st_cnt, t_sum,
      t_cnt.reshape(n_class, 1), s_centroid, t_centroid)
    return loss[0, 0]
